# baseline (speedup 1.0000x reference)
import functools

import jax
import jax.numpy as jnp
import numpy as np
from jax import lax
from jax.experimental import pallas as pl
from jax.experimental.pallas import tpu as pltpu
from jax.experimental.pallas import tpu_sc as plsc

N_NODES = 10000
N_EDGES = 320000
D_FEAT = 128
D_EDGE = 16
UNITS = 128
EPS = 1e-3

NC = 2
NS = 16
NW = NC * NS
CHUNK = 80
N_CHUNKS = 128
EDGES_PER_WORKER = N_CHUNKS * CHUNK
N_EDGES_PAD = NW * EDGES_PER_WORKER
N_PAD = 10240
ROWS_PER_TILE = N_PAD // NS


def _proj_nodes_body(nf_ref, w_ref, out_ref):
    out_ref[...] = jnp.dot(nf_ref[...], w_ref[...],
                           preferred_element_type=jnp.float32)


def _proj_edges_body(ef_ref, w_ref, b_ref, out_ref):
    h = jnp.dot(ef_ref[...], w_ref[...],
                preferred_element_type=jnp.float32) + b_ref[...]
    hb = h.astype(jnp.bfloat16).reshape(h.shape[0] // 2, 2, UNITS)
    lo = jax.lax.bitcast_convert_type(hb[:, 0], jnp.uint16).astype(jnp.uint32)
    hi = jax.lax.bitcast_convert_type(hb[:, 1], jnp.uint16).astype(jnp.uint32)
    out_ref[...] = jax.lax.bitcast_convert_type(lo | (hi << 16), jnp.int32)


def _project(node_features, edge_features, w_node, w_edge, b2d):
    p = pl.pallas_call(
        _proj_nodes_body,
        grid=(5,),
        in_specs=[
            pl.BlockSpec((2000, D_FEAT), lambda i: (i, 0)),
            pl.BlockSpec((D_FEAT, UNITS), lambda i: (0, 0)),
        ],
        out_specs=pl.BlockSpec((2000, UNITS), lambda i: (i, 0)),
        out_shape=jax.ShapeDtypeStruct((N_NODES, UNITS), jnp.float32),
    )(node_features, w_node)

    e = pl.pallas_call(
        _proj_edges_body,
        grid=(40,),
        in_specs=[
            pl.BlockSpec((N_EDGES_PAD // 40, D_EDGE), lambda i: (i, 0)),
            pl.BlockSpec((D_EDGE, UNITS), lambda i: (0, 0)),
            pl.BlockSpec((1, UNITS), lambda i: (0, 0)),
        ],
        out_specs=pl.BlockSpec((N_EDGES_PAD // 80, UNITS), lambda i: (i, 0)),
        out_shape=jax.ShapeDtypeStruct((N_EDGES_PAD // 2, UNITS), jnp.int32),
    )(edge_features, w_edge, b2d)
    return p, e


def _silu_chunk(g_v, e_v, s_v):
    fmt = plsc.PackFormat.INTERLEAVED
    def _pair(rp, _):
        r = 2 * rp
        for q in range(8):
            ew = plsc.bitcast(e_v[rp, pl.ds(16 * q, 16)], jnp.bfloat16)
            ea, eb = plsc.unpack(ew, format=fmt)
            xa = g_v[r, pl.ds(16 * q, 16)] + ea
            xb = g_v[r + 1, pl.ds(16 * q, 16)] + eb
            s_v[r, pl.ds(16 * q, 16)] = xa / (1.0 + jnp.exp(-xa))
            s_v[r + 1, pl.ds(16 * q, 16)] = xb / (1.0 + jnp.exp(-xb))
        return 0
    lax.fori_loop(0, CHUNK // 2, _pair, 0)


def _sc_body(p_hbm, e_hbm, idx3_hbm, part_hbm, acc,
             i0, i1, i2, i3, g0, g1, e0, e1, sbuf,
             gsem0, gsem1, esem0, esem1, isem0, isem1):
    cid = lax.axis_index("c")
    sid = lax.axis_index("s")
    wid = sid * NC + cid
    ibuf = (i0, i1, i2, i3)
    gbuf = (g0, g1)
    ebuf = (e0, e1)
    gsem = (gsem0, gsem1)
    esem = (esem0, esem1)
    isem = (isem0, isem1)

    def _zrow(r, _):
        for j in range(8):
            sbuf[r, pl.ds(j * 16, 16)] = jnp.zeros((16,), jnp.float32)
        return 0
    lax.fori_loop(0, CHUNK, _zrow, 0)
    for k in range(ROWS_PER_TILE // CHUNK):
        pltpu.sync_copy(sbuf, acc.at[pl.ds(sid * ROWS_PER_TILE + k * CHUNK,
                                           CHUNK)])
    plsc.subcore_barrier()

    base2 = wid * (EDGES_PER_WORKER // 2)
    ibase = wid * N_CHUNKS

    def _start(c, s, b):
        pltpu.make_async_copy(idx3_hbm.at[ibase + c], ibuf[s], isem[b]).wait()
        pltpu.async_copy(p_hbm.at[ibuf[s].at[0]], gbuf[b], gsem[b])
        pltpu.async_copy(e_hbm.at[pl.ds(base2 + c * (CHUNK // 2), CHUNK // 2)],
                         ebuf[b], esem[b])

        @pl.when(c + 2 < N_CHUNKS)
        def _():
            pltpu.async_copy(idx3_hbm.at[ibase + c + 2], ibuf[(s + 2) % 4],
                             isem[b])

    def _finish(c, s, b):
        pltpu.make_async_copy(p_hbm.at[ibuf[s].at[0]], gbuf[b],
                              gsem[b]).wait()
        pltpu.make_async_copy(
            e_hbm.at[pl.ds(base2 + c * (CHUNK // 2), CHUNK // 2)],
            ebuf[b], esem[b]).wait()
        _silu_chunk(gbuf[b], ebuf[b], sbuf)
        pltpu.sync_copy(sbuf, acc.at[ibuf[s].at[1]], add=True)

        @pl.when(c + 2 < N_CHUNKS)
        def _():
            _start(c + 2, (s + 2) % 4, b)

    pltpu.async_copy(idx3_hbm.at[ibase], i0, isem0)
    pltpu.async_copy(idx3_hbm.at[ibase + 1], i1, isem1)
    _start(0, 0, 0)
    _start(1, 1, 1)

    def _four(i, _):
        c = 4 * i
        for k in range(4):
            _finish(c + k, k, k % 2)
        return 0
    lax.fori_loop(0, N_CHUNKS // 4, _four, 0)

    plsc.subcore_barrier()
    for k in range(ROWS_PER_TILE // CHUNK):
        r0 = sid * ROWS_PER_TILE + k * CHUNK
        pltpu.sync_copy(acc.at[pl.ds(r0, CHUNK)],
                        part_hbm.at[cid, pl.ds(r0, CHUNK)])


def _sc_aggregate(p, e, idx3):
    mesh = plsc.VectorSubcoreMesh(core_axis_name="c", subcore_axis_name="s")
    f = pl.kernel(
        _sc_body,
        out_type=jax.ShapeDtypeStruct((NC, N_PAD, UNITS), jnp.float32),
        mesh=mesh,
        compiler_params=pltpu.CompilerParams(needs_layout_passes=False),
        scratch_types=(
            [pltpu.VMEM_SHARED((N_PAD, UNITS), jnp.float32)]
            + [pltpu.VMEM((2, CHUNK), jnp.int32)] * 4
            + [pltpu.VMEM((CHUNK, UNITS), jnp.float32)] * 2
            + [pltpu.VMEM((CHUNK // 2, UNITS), jnp.int32)] * 2
            + [pltpu.VMEM((CHUNK, UNITS), jnp.float32)]
            + [pltpu.SemaphoreType.DMA] * 6
        ),
    )
    return f(p, e, idx3)


def _ln_body(part_ref, g_ref, b_ref, out_ref):
    s = part_ref[0] + part_ref[1]
    mean = jnp.mean(s, axis=-1, keepdims=True)
    var = jnp.mean(jnp.square(s - mean), axis=-1, keepdims=True)
    out_ref[...] = (s - mean) * lax.rsqrt(var + EPS) * g_ref[...] + b_ref[...]


def _layernorm(partials, gamma2d, beta2d):
    return pl.pallas_call(
        _ln_body,
        grid=(5,),
        in_specs=[
            pl.BlockSpec((NC, 2000, UNITS), lambda i: (0, i, 0)),
            pl.BlockSpec((1, UNITS), lambda i: (0, 0)),
            pl.BlockSpec((1, UNITS), lambda i: (0, 0)),
        ],
        out_specs=pl.BlockSpec((2000, UNITS), lambda i: (i, 0)),
        out_shape=jax.ShapeDtypeStruct((N_NODES, UNITS), jnp.float32),
    )(partials, gamma2d, beta2d)


@jax.jit
def kernel(node_features, edge_index, edge_features, W, b, gamma, beta):
    n_extra = N_EDGES_PAD - N_EDGES
    idx = edge_index.astype(jnp.int32)
    pad_vals = jnp.stack([jnp.zeros((n_extra,), jnp.int32),
                          jnp.full((n_extra,), N_NODES, jnp.int32)])
    idx3 = (jnp.concatenate([idx, pad_vals], axis=1)
            .reshape(2, NW, N_CHUNKS, CHUNK)
            .transpose(1, 2, 0, 3)
            .reshape(NW * N_CHUNKS, 2, CHUNK))
    ef_pad = jnp.concatenate(
        [edge_features, jnp.zeros((n_extra, D_EDGE), jnp.float32)])
    w_node = W[:D_FEAT]
    w_edge = W[D_FEAT:]
    b2d = b.reshape(1, UNITS)
    p, e = _project(node_features, ef_pad, w_node, w_edge, b2d)
    partials = _sc_aggregate(p, e, idx3)
    return _layernorm(partials, gamma.reshape(1, UNITS),
                      beta.reshape(1, UNITS))


# speedup vs baseline: 2.7636x; 2.7636x over previous
import functools

import jax
import jax.numpy as jnp
from jax import lax
from jax.experimental import pallas as pl
from jax.experimental.pallas import tpu as pltpu
from jax.experimental.pallas import tpu_sc as plsc

N_NODES = 10000
N_EDGES = 320000
D_FEAT = 128
D_EDGE = 16
UNITS = 128
EPS = 1e-3

NC = 2
NS = 16
NW = NC * NS
EDGES_PER_WORKER = N_EDGES // NW
CHUNK = 80
N_CHUNKS = EDGES_PER_WORKER // CHUNK
N_PAD = 10240
ROWS_PER_TILE = N_PAD // NS


def _proj_nodes_body(nf_ref, w_ref, out_ref):
    out_ref[...] = jnp.dot(nf_ref[...], w_ref[...],
                           preferred_element_type=jnp.float32)


def _proj_edges_body(ef_ref, w_ref, b_ref, out_ref):
    out_ref[...] = jnp.dot(ef_ref[...], w_ref[...],
                           preferred_element_type=jnp.float32) + b_ref[...]


def _project(node_features, edge_features, w_node, w_edge, b2d):
    p = pl.pallas_call(
        _proj_nodes_body,
        grid=(5,),
        in_specs=[
            pl.BlockSpec((2000, D_FEAT), lambda i: (i, 0)),
            pl.BlockSpec((D_FEAT, UNITS), lambda i: (0, 0)),
        ],
        out_specs=pl.BlockSpec((2000, UNITS), lambda i: (i, 0)),
        out_shape=jax.ShapeDtypeStruct((N_NODES, UNITS), jnp.float32),
    )(node_features, w_node)

    e = pl.pallas_call(
        _proj_edges_body,
        grid=(40,),
        in_specs=[
            pl.BlockSpec((8000, D_EDGE), lambda i: (i, 0)),
            pl.BlockSpec((D_EDGE, UNITS), lambda i: (0, 0)),
            pl.BlockSpec((1, UNITS), lambda i: (0, 0)),
        ],
        out_specs=pl.BlockSpec((8000, UNITS), lambda i: (i, 0)),
        out_shape=jax.ShapeDtypeStruct((N_EDGES, UNITS), jnp.float32),
    )(edge_features, w_edge, b2d)
    return p, e


def _silu_chunk(g_v, e_v):
    def _row(r, _):
        for j in range(8):
            x = g_v[r, pl.ds(j * 16, 16)] + e_v[r, pl.ds(j * 16, 16)]
            e_v[r, pl.ds(j * 16, 16)] = x / (1.0 + jnp.exp(-x))
        return 0
    lax.fori_loop(0, CHUNK, _row, 0)


def _sc_body(p_hbm, e_hbm, idx3_hbm, part_hbm,
             acc, i0, i1, g0, g1, e0, e1,
             gsem0, gsem1, esem0, esem1):
    cid = lax.axis_index("c")
    sid = lax.axis_index("s")
    wid = sid * NC + cid
    ibuf = (i0, i1)
    gbuf = (g0, g1)
    ebuf = (e0, e1)
    gsem = (gsem0, gsem1)
    esem = (esem0, esem1)

    def _zrow(r, _):
        for j in range(8):
            g0[r, pl.ds(j * 16, 16)] = jnp.zeros((16,), jnp.float32)
        return 0
    lax.fori_loop(0, CHUNK, _zrow, 0)
    for k in range(ROWS_PER_TILE // CHUNK):
        pltpu.sync_copy(g0, acc.at[pl.ds(sid * ROWS_PER_TILE + k * CHUNK,
                                         CHUNK)])
    plsc.subcore_barrier()

    base = wid * EDGES_PER_WORKER

    def _start(c, b):
        pltpu.sync_copy(idx3_hbm.at[wid * N_CHUNKS + c], ibuf[b])
        pltpu.async_copy(p_hbm.at[ibuf[b].at[0]], gbuf[b], gsem[b])
        pltpu.async_copy(e_hbm.at[pl.ds(base + c * CHUNK, CHUNK)],
                         ebuf[b], esem[b])

    def _finish(c, b, prefetch_c):
        pltpu.make_async_copy(p_hbm.at[ibuf[b].at[0]], gbuf[b],
                              gsem[b]).wait()
        pltpu.make_async_copy(e_hbm.at[pl.ds(base + c * CHUNK, CHUNK)],
                              ebuf[b], esem[b]).wait()
        _silu_chunk(gbuf[b], ebuf[b])
        pltpu.sync_copy(ebuf[b], acc.at[ibuf[b].at[1]], add=True)
        if prefetch_c is not None:
            @pl.when(prefetch_c < N_CHUNKS)
            def _():
                _start(prefetch_c, b)

    _start(0, 0)
    _start(1, 1)

    def _pair(i, _):
        c = 2 * i
        _finish(c, 0, c + 2)
        _finish(c + 1, 1, c + 3)
        return 0
    lax.fori_loop(0, N_CHUNKS // 2, _pair, 0)
    _finish(N_CHUNKS - 1, 0, None)

    plsc.subcore_barrier()
    for k in range(ROWS_PER_TILE // CHUNK):
        r0 = sid * ROWS_PER_TILE + k * CHUNK
        pltpu.sync_copy(acc.at[pl.ds(r0, CHUNK)],
                        part_hbm.at[cid, pl.ds(r0, CHUNK)])


def _sc_aggregate(p, e, idx3):
    mesh = plsc.VectorSubcoreMesh(core_axis_name="c", subcore_axis_name="s")
    f = pl.kernel(
        _sc_body,
        out_type=jax.ShapeDtypeStruct((NC, N_PAD, UNITS), jnp.float32),
        mesh=mesh,
        scratch_types=[
            pltpu.VMEM_SHARED((N_PAD, UNITS), jnp.float32),
            pltpu.VMEM((2, CHUNK), jnp.int32),
            pltpu.VMEM((2, CHUNK), jnp.int32),
            pltpu.VMEM((CHUNK, UNITS), jnp.float32),
            pltpu.VMEM((CHUNK, UNITS), jnp.float32),
            pltpu.VMEM((CHUNK, UNITS), jnp.float32),
            pltpu.VMEM((CHUNK, UNITS), jnp.float32),
            pltpu.SemaphoreType.DMA,
            pltpu.SemaphoreType.DMA,
            pltpu.SemaphoreType.DMA,
            pltpu.SemaphoreType.DMA,
        ],
    )
    return f(p, e, idx3)


def _ln_body(part_ref, g_ref, b_ref, out_ref):
    s = part_ref[0] + part_ref[1]
    mean = jnp.mean(s, axis=-1, keepdims=True)
    var = jnp.mean(jnp.square(s - mean), axis=-1, keepdims=True)
    out_ref[...] = (s - mean) * lax.rsqrt(var + EPS) * g_ref[...] + b_ref[...]


def _layernorm(partials, gamma2d, beta2d):
    return pl.pallas_call(
        _ln_body,
        grid=(5,),
        in_specs=[
            pl.BlockSpec((NC, 2000, UNITS), lambda i: (0, i, 0)),
            pl.BlockSpec((1, UNITS), lambda i: (0, 0)),
            pl.BlockSpec((1, UNITS), lambda i: (0, 0)),
        ],
        out_specs=pl.BlockSpec((2000, UNITS), lambda i: (i, 0)),
        out_shape=jax.ShapeDtypeStruct((N_NODES, UNITS), jnp.float32),
    )(partials, gamma2d, beta2d)


@jax.jit
def kernel(node_features, edge_index, edge_features, W, b, gamma, beta):
    idx3 = (edge_index.astype(jnp.int32)
            .reshape(2, NW, N_CHUNKS, CHUNK)
            .transpose(1, 2, 0, 3)
            .reshape(NW * N_CHUNKS, 2, CHUNK))
    w_node = W[:D_FEAT]
    w_edge = W[D_FEAT:]
    p, e = _project(node_features, edge_features, w_node, w_edge,
                    b.reshape(1, UNITS))
    partials = _sc_aggregate(p, e, idx3)
    return _layernorm(partials, gamma.reshape(1, UNITS),
                      beta.reshape(1, UNITS))


# speedup vs baseline: 3.0991x; 1.1214x over previous
import functools

import jax
import jax.numpy as jnp
from jax import lax
from jax.experimental import pallas as pl
from jax.experimental.pallas import tpu as pltpu
from jax.experimental.pallas import tpu_sc as plsc

N_NODES = 10000
N_EDGES = 320000
D_FEAT = 128
D_EDGE = 16
UNITS = 128
EPS = 1e-3

NC = 2
NS = 16
NW = NC * NS
EDGES_PER_WORKER = N_EDGES // NW
CHUNK = 80
N_CHUNKS = EDGES_PER_WORKER // CHUNK
N_PAD = 10240
ROWS_PER_TILE = N_PAD // NS


def _proj_nodes_body(nf_ref, w_ref, out_ref):
    out_ref[...] = jnp.dot(nf_ref[...], w_ref[...],
                           preferred_element_type=jnp.float32)


def _proj_edges_body(ef_ref, w_ref, b_ref, out_ref):
    out_ref[...] = jnp.dot(ef_ref[...], w_ref[...],
                           preferred_element_type=jnp.float32) + b_ref[...]


def _project(node_features, edge_features, w_node, w_edge, b2d):
    p = pl.pallas_call(
        _proj_nodes_body,
        grid=(5,),
        in_specs=[
            pl.BlockSpec((2000, D_FEAT), lambda i: (i, 0)),
            pl.BlockSpec((D_FEAT, UNITS), lambda i: (0, 0)),
        ],
        out_specs=pl.BlockSpec((2000, UNITS), lambda i: (i, 0)),
        out_shape=jax.ShapeDtypeStruct((N_NODES, UNITS), jnp.float32),
    )(node_features, w_node)

    e = pl.pallas_call(
        _proj_edges_body,
        grid=(40,),
        in_specs=[
            pl.BlockSpec((8000, D_EDGE), lambda i: (i, 0)),
            pl.BlockSpec((D_EDGE, UNITS), lambda i: (0, 0)),
            pl.BlockSpec((1, UNITS), lambda i: (0, 0)),
        ],
        out_specs=pl.BlockSpec((8000, UNITS), lambda i: (i, 0)),
        out_shape=jax.ShapeDtypeStruct((N_EDGES, UNITS), jnp.float32),
    )(edge_features, w_edge, b2d)
    return p, e


def _silu_chunk(g_v, e_v):
    def _row(r, _):
        for j in range(8):
            x = g_v[r, pl.ds(j * 16, 16)] + e_v[r, pl.ds(j * 16, 16)]
            e_v[r, pl.ds(j * 16, 16)] = x / (1.0 + jnp.exp(-x))
        return 0
    lax.fori_loop(0, CHUNK, _row, 0)


def _sc_body(p_hbm, e_hbm, idx3_hbm, part_hbm,
             acc, i0, i1, i2, i3, g0, g1, e0, e1,
             gsem0, gsem1, esem0, esem1, isem0, isem1):
    cid = lax.axis_index("c")
    sid = lax.axis_index("s")
    wid = sid * NC + cid
    ibuf = (i0, i1, i2, i3)
    gbuf = (g0, g1)
    ebuf = (e0, e1)
    gsem = (gsem0, gsem1)
    esem = (esem0, esem1)
    isem = (isem0, isem1)

    def _zrow(r, _):
        for j in range(8):
            g0[r, pl.ds(j * 16, 16)] = jnp.zeros((16,), jnp.float32)
        return 0
    lax.fori_loop(0, CHUNK, _zrow, 0)
    for k in range(ROWS_PER_TILE // CHUNK):
        pltpu.sync_copy(g0, acc.at[pl.ds(sid * ROWS_PER_TILE + k * CHUNK,
                                         CHUNK)])
    plsc.subcore_barrier()

    base = wid * EDGES_PER_WORKER
    ibase = wid * N_CHUNKS

    def _start(c, s, b):
        pltpu.make_async_copy(idx3_hbm.at[ibase + c], ibuf[s], isem[b]).wait()
        pltpu.async_copy(p_hbm.at[ibuf[s].at[0]], gbuf[b], gsem[b])
        pltpu.async_copy(e_hbm.at[pl.ds(base + c * CHUNK, CHUNK)],
                         ebuf[b], esem[b])

        @pl.when(c + 2 < N_CHUNKS)
        def _():
            pltpu.async_copy(idx3_hbm.at[ibase + c + 2], ibuf[(s + 2) % 4],
                             isem[b])

    def _finish(c, s, b, prefetch):
        pltpu.make_async_copy(p_hbm.at[ibuf[s].at[0]], gbuf[b],
                              gsem[b]).wait()
        pltpu.make_async_copy(e_hbm.at[pl.ds(base + c * CHUNK, CHUNK)],
                              ebuf[b], esem[b]).wait()
        _silu_chunk(gbuf[b], ebuf[b])
        pltpu.sync_copy(ebuf[b], acc.at[ibuf[s].at[1]], add=True)
        if prefetch:
            @pl.when(c + 2 < N_CHUNKS)
            def _():
                _start(c + 2, (s + 2) % 4, b)

    pltpu.async_copy(idx3_hbm.at[ibase], i0, isem0)
    pltpu.async_copy(idx3_hbm.at[ibase + 1], i1, isem1)
    _start(0, 0, 0)
    _start(1, 1, 1)

    def _four(i, _):
        c = 4 * i
        for k in range(4):
            _finish(c + k, k, k % 2, True)
        return 0
    lax.fori_loop(0, N_CHUNKS // 4, _four, 0)
    _finish(N_CHUNKS - 1, (N_CHUNKS - 1) % 4, (N_CHUNKS - 1) % 2, False)

    plsc.subcore_barrier()
    for k in range(ROWS_PER_TILE // CHUNK):
        r0 = sid * ROWS_PER_TILE + k * CHUNK
        pltpu.sync_copy(acc.at[pl.ds(r0, CHUNK)],
                        part_hbm.at[cid, pl.ds(r0, CHUNK)])


def _sc_aggregate(p, e, idx3):
    mesh = plsc.VectorSubcoreMesh(core_axis_name="c", subcore_axis_name="s")
    f = pl.kernel(
        _sc_body,
        out_type=jax.ShapeDtypeStruct((NC, N_PAD, UNITS), jnp.float32),
        mesh=mesh,
        scratch_types=[
            pltpu.VMEM_SHARED((N_PAD, UNITS), jnp.float32),
            pltpu.VMEM((2, CHUNK), jnp.int32),
            pltpu.VMEM((2, CHUNK), jnp.int32),
            pltpu.VMEM((2, CHUNK), jnp.int32),
            pltpu.VMEM((2, CHUNK), jnp.int32),
            pltpu.VMEM((CHUNK, UNITS), jnp.float32),
            pltpu.VMEM((CHUNK, UNITS), jnp.float32),
            pltpu.VMEM((CHUNK, UNITS), jnp.float32),
            pltpu.VMEM((CHUNK, UNITS), jnp.float32),
            pltpu.SemaphoreType.DMA,
            pltpu.SemaphoreType.DMA,
            pltpu.SemaphoreType.DMA,
            pltpu.SemaphoreType.DMA,
            pltpu.SemaphoreType.DMA,
            pltpu.SemaphoreType.DMA,
        ],
    )
    return f(p, e, idx3)


def _ln_body(part_ref, g_ref, b_ref, out_ref):
    s = part_ref[0] + part_ref[1]
    mean = jnp.mean(s, axis=-1, keepdims=True)
    var = jnp.mean(jnp.square(s - mean), axis=-1, keepdims=True)
    out_ref[...] = (s - mean) * lax.rsqrt(var + EPS) * g_ref[...] + b_ref[...]


def _layernorm(partials, gamma2d, beta2d):
    return pl.pallas_call(
        _ln_body,
        grid=(5,),
        in_specs=[
            pl.BlockSpec((NC, 2000, UNITS), lambda i: (0, i, 0)),
            pl.BlockSpec((1, UNITS), lambda i: (0, 0)),
            pl.BlockSpec((1, UNITS), lambda i: (0, 0)),
        ],
        out_specs=pl.BlockSpec((2000, UNITS), lambda i: (i, 0)),
        out_shape=jax.ShapeDtypeStruct((N_NODES, UNITS), jnp.float32),
    )(partials, gamma2d, beta2d)


@jax.jit
def kernel(node_features, edge_index, edge_features, W, b, gamma, beta):
    idx3 = (edge_index.astype(jnp.int32)
            .reshape(2, NW, N_CHUNKS, CHUNK)
            .transpose(1, 2, 0, 3)
            .reshape(NW * N_CHUNKS, 2, CHUNK))
    w_node = W[:D_FEAT]
    w_edge = W[D_FEAT:]
    p, e = _project(node_features, edge_features, w_node, w_edge,
                    b.reshape(1, UNITS))
    partials = _sc_aggregate(p, e, idx3)
    return _layernorm(partials, gamma.reshape(1, UNITS),
                      beta.reshape(1, UNITS))


# speedup vs baseline: 3.1169x; 1.0057x over previous
import functools

import jax
import jax.numpy as jnp
from jax import lax
from jax.experimental import pallas as pl
from jax.experimental.pallas import tpu as pltpu
from jax.experimental.pallas import tpu_sc as plsc

N_NODES = 10000
N_EDGES = 320000
D_FEAT = 128
D_EDGE = 16
UNITS = 128
EPS = 1e-3

NC = 2
NS = 16
NW = NC * NS
EDGES_PER_WORKER = N_EDGES // NW
CHUNK = 80
N_CHUNKS = EDGES_PER_WORKER // CHUNK
N_PAD = 10240
ROWS_PER_TILE = N_PAD // NS


def _proj_body(nf_ref, ef_ref, wn_ref, we_ref, b_ref, p_ref, e_ref):
    i = pl.program_id(0)

    @pl.when(i == 0)
    def _():
        p_ref[...] = jnp.dot(nf_ref[...], wn_ref[...],
                             preferred_element_type=jnp.float32)

    @pl.when(i > 0)
    def _():
        e_ref[...] = jnp.dot(ef_ref[...], we_ref[...],
                             preferred_element_type=jnp.float32) + b_ref[...]


def _project(node_features, edge_features, w_node, w_edge, b2d):
    eblk = lambda i: (jnp.maximum(i - 1, 0), 0)
    return pl.pallas_call(
        _proj_body,
        grid=(41,),
        in_specs=[
            pl.BlockSpec((N_NODES, D_FEAT), lambda i: (0, 0)),
            pl.BlockSpec((8000, D_EDGE), eblk),
            pl.BlockSpec((D_FEAT, UNITS), lambda i: (0, 0)),
            pl.BlockSpec((D_EDGE, UNITS), lambda i: (0, 0)),
            pl.BlockSpec((1, UNITS), lambda i: (0, 0)),
        ],
        out_specs=[
            pl.BlockSpec((N_NODES, UNITS), lambda i: (0, 0)),
            pl.BlockSpec((8000, UNITS), eblk),
        ],
        out_shape=[
            jax.ShapeDtypeStruct((N_NODES, UNITS), jnp.float32),
            jax.ShapeDtypeStruct((N_EDGES, UNITS), jnp.float32),
        ],
    )(node_features, edge_features, w_node, w_edge, b2d)


def _silu_chunk(g_v, e_v):
    def _row(r, _):
        for j in range(8):
            x = g_v[r, pl.ds(j * 16, 16)] + e_v[r, pl.ds(j * 16, 16)]
            e_v[r, pl.ds(j * 16, 16)] = x / (1.0 + jnp.exp(-x))
        return 0
    lax.fori_loop(0, CHUNK, _row, 0)


def _sc_body(p_hbm, e_hbm, idx3_hbm, part_hbm,
             acc, i0, i1, i2, i3, g0, g1, e0, e1,
             gsem0, gsem1, esem0, esem1, isem0, isem1):
    cid = lax.axis_index("c")
    sid = lax.axis_index("s")
    wid = sid * NC + cid
    ibuf = (i0, i1, i2, i3)
    gbuf = (g0, g1)
    ebuf = (e0, e1)
    gsem = (gsem0, gsem1)
    esem = (esem0, esem1)
    isem = (isem0, isem1)

    def _zrow(r, _):
        for j in range(8):
            g0[r, pl.ds(j * 16, 16)] = jnp.zeros((16,), jnp.float32)
        return 0
    lax.fori_loop(0, CHUNK, _zrow, 0)
    for k in range(ROWS_PER_TILE // CHUNK):
        pltpu.sync_copy(g0, acc.at[pl.ds(sid * ROWS_PER_TILE + k * CHUNK,
                                         CHUNK)])
    plsc.subcore_barrier()

    base = wid * EDGES_PER_WORKER
    ibase = wid * N_CHUNKS

    def _start(c, s, b):
        pltpu.make_async_copy(idx3_hbm.at[ibase + c], ibuf[s], isem[b]).wait()
        pltpu.async_copy(p_hbm.at[ibuf[s].at[0]], gbuf[b], gsem[b])
        pltpu.async_copy(e_hbm.at[pl.ds(base + c * CHUNK, CHUNK)],
                         ebuf[b], esem[b])

        @pl.when(c + 2 < N_CHUNKS)
        def _():
            pltpu.async_copy(idx3_hbm.at[ibase + c + 2], ibuf[(s + 2) % 4],
                             isem[b])

    def _finish(c, s, b, prefetch):
        pltpu.make_async_copy(p_hbm.at[ibuf[s].at[0]], gbuf[b],
                              gsem[b]).wait()
        pltpu.make_async_copy(e_hbm.at[pl.ds(base + c * CHUNK, CHUNK)],
                              ebuf[b], esem[b]).wait()
        _silu_chunk(gbuf[b], ebuf[b])
        pltpu.sync_copy(ebuf[b], acc.at[ibuf[s].at[1]], add=True)
        if prefetch:
            @pl.when(c + 2 < N_CHUNKS)
            def _():
                _start(c + 2, (s + 2) % 4, b)

    pltpu.async_copy(idx3_hbm.at[ibase], i0, isem0)
    pltpu.async_copy(idx3_hbm.at[ibase + 1], i1, isem1)
    _start(0, 0, 0)
    _start(1, 1, 1)

    def _four(i, _):
        c = 4 * i
        for k in range(4):
            _finish(c + k, k, k % 2, True)
        return 0
    lax.fori_loop(0, N_CHUNKS // 4, _four, 0)
    _finish(N_CHUNKS - 1, (N_CHUNKS - 1) % 4, (N_CHUNKS - 1) % 2, False)

    plsc.subcore_barrier()
    for k in range(ROWS_PER_TILE // CHUNK):
        r0 = sid * ROWS_PER_TILE + k * CHUNK
        pltpu.sync_copy(acc.at[pl.ds(r0, CHUNK)],
                        part_hbm.at[cid, pl.ds(r0, CHUNK)])


def _sc_aggregate(p, e, idx3):
    mesh = plsc.VectorSubcoreMesh(core_axis_name="c", subcore_axis_name="s")
    f = pl.kernel(
        _sc_body,
        out_type=jax.ShapeDtypeStruct((NC, N_PAD, UNITS), jnp.float32),
        mesh=mesh,
        scratch_types=[
            pltpu.VMEM_SHARED((N_PAD, UNITS), jnp.float32),
            pltpu.VMEM((2, CHUNK), jnp.int32),
            pltpu.VMEM((2, CHUNK), jnp.int32),
            pltpu.VMEM((2, CHUNK), jnp.int32),
            pltpu.VMEM((2, CHUNK), jnp.int32),
            pltpu.VMEM((CHUNK, UNITS), jnp.float32),
            pltpu.VMEM((CHUNK, UNITS), jnp.float32),
            pltpu.VMEM((CHUNK, UNITS), jnp.float32),
            pltpu.VMEM((CHUNK, UNITS), jnp.float32),
            pltpu.SemaphoreType.DMA,
            pltpu.SemaphoreType.DMA,
            pltpu.SemaphoreType.DMA,
            pltpu.SemaphoreType.DMA,
            pltpu.SemaphoreType.DMA,
            pltpu.SemaphoreType.DMA,
        ],
    )
    return f(p, e, idx3)


def _ln_body(part_ref, g_ref, b_ref, out_ref):
    s = part_ref[0] + part_ref[1]
    mean = jnp.mean(s, axis=-1, keepdims=True)
    var = jnp.mean(jnp.square(s - mean), axis=-1, keepdims=True)
    out_ref[...] = (s - mean) * lax.rsqrt(var + EPS) * g_ref[...] + b_ref[...]


def _layernorm(partials, gamma2d, beta2d):
    return pl.pallas_call(
        _ln_body,
        grid=(5,),
        in_specs=[
            pl.BlockSpec((NC, 2000, UNITS), lambda i: (0, i, 0)),
            pl.BlockSpec((1, UNITS), lambda i: (0, 0)),
            pl.BlockSpec((1, UNITS), lambda i: (0, 0)),
        ],
        out_specs=pl.BlockSpec((2000, UNITS), lambda i: (i, 0)),
        out_shape=jax.ShapeDtypeStruct((N_NODES, UNITS), jnp.float32),
    )(partials, gamma2d, beta2d)


@jax.jit
def kernel(node_features, edge_index, edge_features, W, b, gamma, beta):
    idx3 = (edge_index.astype(jnp.int32)
            .reshape(2, NW, N_CHUNKS, CHUNK)
            .transpose(1, 2, 0, 3)
            .reshape(NW * N_CHUNKS, 2, CHUNK))
    w_node = W[:D_FEAT]
    w_edge = W[D_FEAT:]
    p, e = _project(node_features, edge_features, w_node, w_edge,
                    b.reshape(1, UNITS))
    partials = _sc_aggregate(p, e, idx3)
    return _layernorm(partials, gamma.reshape(1, UNITS),
                      beta.reshape(1, UNITS))
